# Initial kernel scaffold; baseline (speedup 1.0000x reference)
import jax
import jax.numpy as jnp
from jax.experimental import pallas as pl


def kernel(x, pe):
    raise NotImplementedError("write your pallas kernel here")


# speedup vs baseline: 1.8765x; 1.8765x over previous
import jax
import jax.numpy as jnp
from jax.experimental import pallas as pl

_SBLK = 1024


def _add_pe_block(x_ref, pe_ref, o_ref):
    o_ref[...] = x_ref[...] + pe_ref[...]


def kernel(x, pe):
    B, S, D = x.shape
    n_s = pl.cdiv(S, _SBLK)
    return pl.pallas_call(
        _add_pe_block,
        grid=(n_s, B),
        in_specs=[
            pl.BlockSpec((1, _SBLK, D), lambda s, b: (b, s, 0)),
            pl.BlockSpec((_SBLK, D), lambda s, b: (s, 0)),
        ],
        out_specs=pl.BlockSpec((1, _SBLK, D), lambda s, b: (b, s, 0)),
        out_shape=jax.ShapeDtypeStruct(x.shape, x.dtype),
    )(x, pe)


# speedup vs baseline: 1.9982x; 1.0649x over previous
import jax
import jax.numpy as jnp
from jax.experimental import pallas as pl

_SBLK = 2048


def _add_pe_block(x_ref, pe_ref, o_ref):
    o_ref[...] = x_ref[...] + pe_ref[...]


def kernel(x, pe):
    B, S, D = x.shape
    n_s = pl.cdiv(S, _SBLK)
    return pl.pallas_call(
        _add_pe_block,
        grid=(n_s, B),
        in_specs=[
            pl.BlockSpec((1, _SBLK, D), lambda s, b: (b, s, 0)),
            pl.BlockSpec((_SBLK, D), lambda s, b: (s, 0)),
        ],
        out_specs=pl.BlockSpec((1, _SBLK, D), lambda s, b: (b, s, 0)),
        out_shape=jax.ShapeDtypeStruct(x.shape, x.dtype),
    )(x, pe)
